# baseline (speedup 1.0000x reference)
import functools
import jax
import jax.numpy as jnp
from jax import lax
from jax.experimental import pallas as pl
from jax.experimental.pallas import tpu as pltpu
from jax.experimental.pallas import tpu_sc as plsc

N_REAL = 6890
NPAD = 6912
B = 4
CHALF = 256
ROW = B * CHALF
NCHUNK = 8
CW = ROW // NCHUNK
NSC = 2
NSUB = 16
G = 128
RPT = NPAD // NSUB
TN = 768
NT = NPAD // TN
EPS = 1e-5

HI = lax.Precision.HIGHEST


def _mm(a, b, dims):
    return lax.dot_general(a, b, (dims, ((), ())), precision=HI,
                           preferred_element_type=jnp.float32)


def _gn_rows(stats, gamma, beta):
    C = stats.shape[0]
    ng = C // 8
    gi = lax.broadcasted_iota(jnp.int32, (ng, C), 1) // 8
    gj = lax.broadcasted_iota(jnp.int32, (ng, C), 0)
    Gm = (gi == gj).astype(jnp.float32)
    gs = _mm(Gm, stats, (((1,), (0,))))
    denom = 8.0 * N_REAL
    mean = gs[:, 0:1] / denom
    var = gs[:, 1:2] / denom - mean * mean
    rstd = lax.rsqrt(var + EPS)
    mean_c = _mm(Gm, mean, (((0,), (0,))))
    rstd_c = _mm(Gm, rstd, (((0,), (0,))))
    scale = rstd_c * gamma
    shift = beta - mean_c * scale
    return scale, shift


def _gn_lanes(srow, sqrow, gamma, beta):
    Cl = srow.shape[1]
    ng = Cl // 8
    gi = lax.broadcasted_iota(jnp.int32, (Cl, ng), 0) // 8
    gj = lax.broadcasted_iota(jnp.int32, (Cl, ng), 1)
    Gm = (gi == gj).astype(jnp.float32)
    denom = 8.0 * N_REAL
    gmean = _mm(srow, Gm, (((1,), (0,)))) / denom
    gmsq = _mm(sqrow, Gm, (((1,), (0,)))) / denom
    var = gmsq - gmean * gmean
    rstd = lax.rsqrt(var + EPS)
    mean_c = _mm(gmean, Gm, (((1,), (1,))))
    rstd_c = _mm(rstd, Gm, (((1,), (1,))))
    scale = rstd_c * gamma
    shift = beta - mean_c * scale
    return scale, shift


def _mask_lanes(n):
    col = n * TN + lax.broadcasted_iota(jnp.int32, (1, TN), 1)
    return (col < N_REAL).astype(jnp.float32)


def _acc_stats(ref, n, y, mask):
    s = jnp.sum(y * mask, axis=1, keepdims=True)
    q = jnp.sum(y * y * mask, axis=1, keepdims=True)
    contrib = jnp.concatenate([s, q], axis=1)

    @pl.when(n == 0)
    def _():
        ref[0] = jnp.zeros_like(ref[0])

    ref[0] += contrib


_SEQ = pltpu.CompilerParams(dimension_semantics=("arbitrary", "arbitrary"))
_SEQ1 = pltpu.CompilerParams(dimension_semantics=("arbitrary",))


def _k_enc(Wenc, img, b0):
    def body(w_ref, i_ref, b_ref, o_ref):
        o_ref[...] = _mm(w_ref[...], i_ref[...], (((1,), (1,)))) + b_ref[...]
    return pl.pallas_call(
        body, grid=(1,),
        in_specs=[pl.BlockSpec(Wenc.shape, lambda i: (0, 0)),
                  pl.BlockSpec(img.shape, lambda i: (0, 0)),
                  pl.BlockSpec((Wenc.shape[0], 1), lambda i: (0, 0))],
        out_specs=pl.BlockSpec((Wenc.shape[0], B), lambda i: (0, 0)),
        out_shape=jax.ShapeDtypeStruct((Wenc.shape[0], B), jnp.float32),
    )(Wenc, img, b0)


def _k_lin0(Wrv8, rv8, yenc):
    C = Wrv8.shape[0]

    def body(w_ref, r_ref, e_ref, o_ref, st_ref):
        n = pl.program_id(1)
        y = _mm(w_ref[...], r_ref[...], (((1,), (0,)))) + e_ref[0]
        o_ref[0] = y
        _acc_stats(st_ref, n, y, _mask_lanes(n))

    return pl.pallas_call(
        body, grid=(B, NT),
        in_specs=[pl.BlockSpec((C, 8), lambda b, n: (0, 0)),
                  pl.BlockSpec((8, TN), lambda b, n: (0, n)),
                  pl.BlockSpec((1, C, 1), lambda b, n: (b, 0, 0))],
        out_specs=[pl.BlockSpec((1, C, TN), lambda b, n: (b, 0, n)),
                   pl.BlockSpec((1, C, 2), lambda b, n: (b, 0, 0))],
        out_shape=[jax.ShapeDtypeStruct((B, C, NPAD), jnp.float32),
                   jax.ShapeDtypeStruct((B, C, 2), jnp.float32)],
        compiler_params=_SEQ,
    )(Wrv8, rv8, yenc)


def _k_a(x, xstats, gamma, beta, W, bias):
    Cin = W.shape[1]
    Cout = W.shape[0]

    def body(x_ref, st_ref, g_ref, be_ref, w_ref, b_ref, o_ref, ost_ref):
        n = pl.program_id(1)
        scale, shift = _gn_rows(st_ref[0], g_ref[...], be_ref[...])
        xn = jax.nn.relu(x_ref[0] * scale + shift)
        y = _mm(w_ref[...], xn, (((1,), (0,)))) + b_ref[...]
        o_ref[0] = y
        _acc_stats(ost_ref, n, y, _mask_lanes(n))

    return pl.pallas_call(
        body, grid=(B, NT),
        in_specs=[pl.BlockSpec((1, Cin, TN), lambda b, n: (b, 0, n)),
                  pl.BlockSpec((1, Cin, 2), lambda b, n: (b, 0, 0)),
                  pl.BlockSpec((Cin, 1), lambda b, n: (0, 0)),
                  pl.BlockSpec((Cin, 1), lambda b, n: (0, 0)),
                  pl.BlockSpec((Cout, Cin), lambda b, n: (0, 0)),
                  pl.BlockSpec((Cout, 1), lambda b, n: (0, 0))],
        out_specs=[pl.BlockSpec((1, Cout, TN), lambda b, n: (b, 0, n)),
                   pl.BlockSpec((1, Cout, 2), lambda b, n: (b, 0, 0))],
        out_shape=[jax.ShapeDtypeStruct((B, Cout, NPAD), jnp.float32),
                   jax.ShapeDtypeStruct((B, Cout, 2), jnp.float32)],
        compiler_params=_SEQ,
    )(x, xstats, gamma, beta, W, bias)


def _k_b(y1, y1stats, gamma, beta, Wc):
    def body(x_ref, st_ref, g_ref, be_ref, w_ref, o_ref):
        scale, shift = _gn_rows(st_ref[0], g_ref[...], be_ref[...])
        xn = jax.nn.relu(x_ref[0] * scale + shift)
        h = _mm(xn, w_ref[...], (((0,), (0,))))
        o_ref[0] = h[:, :CW]
        o_ref[1] = h[:, CW:]

    return pl.pallas_call(
        body, grid=(B, NT),
        in_specs=[pl.BlockSpec((1, CHALF, TN), lambda b, n: (b, 0, n)),
                  pl.BlockSpec((1, CHALF, 2), lambda b, n: (b, 0, 0)),
                  pl.BlockSpec((CHALF, 1), lambda b, n: (0, 0)),
                  pl.BlockSpec((CHALF, 1), lambda b, n: (0, 0)),
                  pl.BlockSpec((CHALF, CHALF), lambda b, n: (0, 0))],
        out_specs=pl.BlockSpec((2, TN, CW), lambda b, n: (b, n, 0)),
        out_shape=jax.ShapeDtypeStruct((NCHUNK, NPAD, CW), jnp.float32),
        compiler_params=_SEQ,
    )(y1, y1stats, gamma, beta, Wc)


def _k_stats(s):
    def body(s_ref, o_ref):
        n = pl.program_id(0)

        @pl.when(n == 0)
        def _():
            o_ref[...] = jnp.zeros_like(o_ref)

        for c in range(NCHUNK):
            sc = s_ref[c]
            o_ref[c, 0:1, :] += jnp.sum(sc, axis=0, keepdims=True)
            o_ref[c, 1:2, :] += jnp.sum(sc * sc, axis=0, keepdims=True)

    return pl.pallas_call(
        body, grid=(NT,),
        in_specs=[pl.BlockSpec((NCHUNK, TN, CW), lambda n: (0, n, 0))],
        out_specs=pl.BlockSpec((NCHUNK, 2, CW), lambda n: (0, 0, 0)),
        out_shape=jax.ShapeDtypeStruct((NCHUNK, 2, CW), jnp.float32),
        compiler_params=_SEQ1,
    )(s)


def _k_c(s, sstats, conv_b, gamma, beta, W2, b2, x, skip_W, skip_b):
    Cin = x.shape[1]
    Cout = W2.shape[0]
    has_skip = skip_W is not None

    def body(*refs):
        if has_skip:
            (s_ref, st_ref, cb_ref, g_ref, be_ref, w2_ref, b2_ref,
             x_ref, sw_ref, sb_ref, o_ref, ost_ref) = refs
        else:
            (s_ref, st_ref, cb_ref, g_ref, be_ref, w2_ref, b2_ref,
             x_ref, o_ref, ost_ref) = refs
        n = pl.program_id(1)
        z = b2_ref[...] * jnp.ones((1, TN), jnp.float32)
        for q in range(2):
            raw_s = st_ref[q, 0:1, :]
            raw_q = st_ref[q, 1:2, :]
            cb = cb_ref[q:q + 1, :]
            s_adj = raw_s + N_REAL * cb
            q_adj = raw_q + 2.0 * cb * raw_s + N_REAL * cb * cb
            scale, shift = _gn_lanes(s_adj, q_adj,
                                     g_ref[q:q + 1, :], be_ref[q:q + 1, :])
            sq = s_ref[q] + cb
            xq = jax.nn.relu(sq * scale + shift)
            z = z + _mm(w2_ref[:, q * CW:(q + 1) * CW], xq, (((1,), (1,))))
        if has_skip:
            z = z + _mm(sw_ref[...], x_ref[0], (((1,), (0,)))) + sb_ref[...]
        else:
            z = z + x_ref[0]
        o_ref[0] = z
        _acc_stats(ost_ref, n, z, _mask_lanes(n))

    in_arrays = [s, sstats, conv_b, gamma, beta, W2, b2, x]
    in_specs = [pl.BlockSpec((2, TN, CW), lambda b, n: (b, n, 0)),
                pl.BlockSpec((2, 2, CW), lambda b, n: (b, 0, 0)),
                pl.BlockSpec((2, CW), lambda b, n: (0, 0)),
                pl.BlockSpec((2, CW), lambda b, n: (0, 0)),
                pl.BlockSpec((2, CW), lambda b, n: (0, 0)),
                pl.BlockSpec((Cout, CHALF), lambda b, n: (0, 0)),
                pl.BlockSpec((Cout, 1), lambda b, n: (0, 0)),
                pl.BlockSpec((1, Cin, TN), lambda b, n: (b, 0, n))]
    if has_skip:
        in_arrays += [skip_W, skip_b]
        in_specs += [pl.BlockSpec((Cout, Cin), lambda b, n: (0, 0)),
                     pl.BlockSpec((Cout, 1), lambda b, n: (0, 0))]

    return pl.pallas_call(
        body, grid=(B, NT),
        in_specs=in_specs,
        out_specs=[pl.BlockSpec((1, Cout, TN), lambda b, n: (b, 0, n)),
                   pl.BlockSpec((1, Cout, 2), lambda b, n: (b, 0, 0))],
        out_shape=[jax.ShapeDtypeStruct((B, Cout, NPAD), jnp.float32),
                   jax.ShapeDtypeStruct((B, Cout, 2), jnp.float32)],
        compiler_params=_SEQ,
    )(*in_arrays)


def _k_d(y, W1, b1, W2, b2):
    C1 = W1.shape[0]
    C2 = W2.shape[0]
    Cin = W1.shape[1]

    def body(y_ref, w1_ref, b1_ref, w2_ref, b2_ref, o_ref, ost_ref):
        n = pl.program_id(1)
        t = jax.nn.relu(_mm(w1_ref[...], y_ref[0], (((1,), (0,)))) + b1_ref[...])
        u = _mm(w2_ref[...], t, (((1,), (0,)))) + b2_ref[...]
        o_ref[0] = u
        _acc_stats(ost_ref, n, u, _mask_lanes(n))

    return pl.pallas_call(
        body, grid=(B, NT),
        in_specs=[pl.BlockSpec((1, Cin, TN), lambda b, n: (b, 0, n)),
                  pl.BlockSpec((C1, Cin), lambda b, n: (0, 0)),
                  pl.BlockSpec((C1, 1), lambda b, n: (0, 0)),
                  pl.BlockSpec((C2, C1), lambda b, n: (0, 0)),
                  pl.BlockSpec((C2, 1), lambda b, n: (0, 0))],
        out_specs=[pl.BlockSpec((1, C2, TN), lambda b, n: (b, 0, n)),
                   pl.BlockSpec((1, C2, 2), lambda b, n: (b, 0, 0))],
        out_shape=[jax.ShapeDtypeStruct((B, C2, NPAD), jnp.float32),
                   jax.ShapeDtypeStruct((B, C2, 2), jnp.float32)],
        compiler_params=_SEQ,
    )(y, W1, b1, W2, b2)


def _k_e(u, ustats, gamma, beta, W3, b3):
    Cin = W3.shape[1]
    Cout = W3.shape[0]

    def body(u_ref, st_ref, g_ref, be_ref, w_ref, b_ref, o_ref):
        scale, shift = _gn_rows(st_ref[0], g_ref[...], be_ref[...])
        xn = jax.nn.relu(u_ref[0] * scale + shift)
        o_ref[0] = _mm(w_ref[...], xn, (((1,), (0,)))) + b_ref[...]

    return pl.pallas_call(
        body, grid=(B, NT),
        in_specs=[pl.BlockSpec((1, Cin, TN), lambda b, n: (b, 0, n)),
                  pl.BlockSpec((1, Cin, 2), lambda b, n: (b, 0, 0)),
                  pl.BlockSpec((Cin, 1), lambda b, n: (0, 0)),
                  pl.BlockSpec((Cin, 1), lambda b, n: (0, 0)),
                  pl.BlockSpec((Cout, Cin), lambda b, n: (0, 0)),
                  pl.BlockSpec((Cout, 1), lambda b, n: (0, 0))],
        out_specs=pl.BlockSpec((1, Cout, TN), lambda b, n: (b, 0, n)),
        out_shape=jax.ShapeDtypeStruct((B, Cout, NPAD), jnp.float32),
        compiler_params=_SEQ,
    )(u, ustats, gamma, beta, W3, b3)


def _spmm_sc(h4, src4, dstv, w16, n_batches):
    epad = dstv.shape[0]
    ept = epad // NSUB

    mesh = plsc.VectorSubcoreMesh(core_axis_name="c", subcore_axis_name="s",
                                  num_cores=NSC, num_subcores=NSUB)

    @functools.partial(
        pl.kernel, mesh=mesh,
        out_type=jax.ShapeDtypeStruct((NCHUNK * NPAD, CW), jnp.float32),
        scratch_types=[
            pltpu.VMEM((G,), jnp.int32),
            pltpu.VMEM((G,), jnp.int32),
            pltpu.VMEM((G, 16), jnp.float32),
            pltpu.VMEM((G, CW), jnp.float32),
            pltpu.VMEM((16, CW), jnp.float32),
            pltpu.VMEM_SHARED((NPAD, CW), jnp.float32),
            pltpu.SemaphoreType.DMA,
        ],
    )
    def k(h4_hbm, src4_hbm, dstv_hbm, w16_hbm, out_hbm,
          idx_v, dst_v, w_v, rows_v, zero_v, acc, sem):
        core = lax.axis_index("c")
        sub = lax.axis_index("s")
        zvec = jnp.zeros((16,), jnp.float32)
        for r in range(16):
            for kk in range(CW // 16):
                zero_v[r, pl.ds(kk * 16, 16)] = zvec
        for cc in range(NCHUNK // NSC):
            chunk = core * (NCHUNK // NSC) + cc
            for z in range(RPT // 16):
                pltpu.sync_copy(zero_v, acc.at[pl.ds(sub * RPT + z * 16, 16)])
            plsc.subcore_barrier()

            def batch_body(b, _):
                off = sub * ept + b * G
                pltpu.sync_copy(src4_hbm.at[pl.ds(chunk * epad + off, G)], idx_v)
                pltpu.sync_copy(dstv_hbm.at[pl.ds(off, G)], dst_v)
                pltpu.sync_copy(w16_hbm.at[pl.ds(off, G)], w_v)
                pltpu.async_copy(h4_hbm.at[idx_v], rows_v, sem).wait()

                def g_body(g, _):
                    wv = w_v[g, :]
                    for kk in range(CW // 16):
                        sl = pl.ds(kk * 16, 16)
                        rows_v[g, sl] = rows_v[g, sl] * wv
                    return 0

                lax.fori_loop(0, G, g_body, 0)
                pltpu.sync_copy(rows_v, acc.at[dst_v], add=True)
                return 0

            lax.fori_loop(0, n_batches, batch_body, 0)
            plsc.subcore_barrier()
            pltpu.sync_copy(
                acc.at[pl.ds(sub * RPT, RPT)],
                out_hbm.at[pl.ds(chunk * NPAD + sub * RPT, RPT)])
            plsc.subcore_barrier()

    return k(h4, src4, dstv, w16)


def kernel(image_resnet, params, ref_vertices, edge_index, edge_weight):
    N = ref_vertices.shape[1]
    f32 = jnp.float32

    src = edge_index[0]
    dst = edge_index[1]
    E = src.shape[0]
    epad = ((E + NSUB * G - 1) // (NSUB * G)) * (NSUB * G)
    n_batches = epad // (NSUB * G)
    pad = epad - E
    src_p = jnp.pad(src, (0, pad))
    dst_p = jnp.pad(dst, (0, pad))
    w_p = jnp.pad(edge_weight, (0, pad))
    src4 = (src_p[None, :]
            + NPAD * jnp.arange(NCHUNK, dtype=jnp.int32)[:, None]).reshape(-1)
    w16 = jnp.broadcast_to(w_p[:, None], (epad, 16))

    W0 = params['lin0_W']
    rv8 = jnp.pad(ref_vertices, ((0, 5), (0, NPAD - N)))
    Wrv8 = jnp.pad(W0[:, :3], ((0, 0), (0, 5)))
    yenc = _k_enc(W0[:, 3:], image_resnet, params['lin0_b'][:, None])
    yenc = jnp.transpose(yenc)[:, :, None]
    x, xstats = _k_lin0(Wrv8, rv8, yenc)

    for p in params['blocks']:
        cv = lambda a: a[:, None].astype(f32)
        y1, y1stats = _k_a(x, xstats, cv(p['pre_g']), cv(p['pre_b']),
                           p['lin1_W'], cv(p['lin1_b']))
        h = _k_b(y1, y1stats, cv(p['n1_g']), cv(p['n1_b']), p['conv_W'])
        s4 = _spmm_sc(h.reshape(NCHUNK * NPAD, CW), src4, dst_p, w16,
                      n_batches)
        s = s4.reshape(NCHUNK, NPAD, CW)
        sstats = _k_stats(s)
        x, xstats = _k_c(s, sstats, p['conv_b'].reshape(2, CW),
                         p['n2_g'].reshape(2, CW), p['n2_b'].reshape(2, CW),
                         p['lin2_W'], cv(p['lin2_b']), x,
                         p.get('skip_W'), cv(p['skip_b']) if 'skip_W' in p else None)

    u, ustats = _k_d(x, params['shape_W1'], cv(params['shape_b1']),
                     params['shape_W2'], cv(params['shape_b2']))
    out = _k_e(u, ustats, cv(params['shape_ng']), cv(params['shape_nb']),
               params['shape_W3'], cv(params['shape_b3']))
    return out[:, :, :N]


# speedup vs baseline: 1.1408x; 1.1408x over previous
import functools
import jax
import jax.numpy as jnp
from jax import lax
from jax.experimental import pallas as pl
from jax.experimental.pallas import tpu as pltpu
from jax.experimental.pallas import tpu_sc as plsc

N_REAL = 6890
NPAD = 6912
B = 4
CHALF = 256
ROW = B * CHALF
NCHUNK = 8
CW = ROW // NCHUNK
NSC = 2
NSUB = 16
G = 128
RPT = NPAD // NSUB
TN = 768
NT = NPAD // TN
EPS = 1e-5

HI = lax.Precision.HIGHEST


def _mm(a, b, dims):
    return lax.dot_general(a, b, (dims, ((), ())), precision=HI,
                           preferred_element_type=jnp.float32)


def _gn_rows(stats, gamma, beta):
    C = stats.shape[0]
    ng = C // 8
    gi = lax.broadcasted_iota(jnp.int32, (ng, C), 1) // 8
    gj = lax.broadcasted_iota(jnp.int32, (ng, C), 0)
    Gm = (gi == gj).astype(jnp.float32)
    gs = _mm(Gm, stats, (((1,), (0,))))
    denom = 8.0 * N_REAL
    mean = gs[:, 0:1] / denom
    var = gs[:, 1:2] / denom - mean * mean
    rstd = lax.rsqrt(var + EPS)
    mean_c = _mm(Gm, mean, (((0,), (0,))))
    rstd_c = _mm(Gm, rstd, (((0,), (0,))))
    scale = rstd_c * gamma
    shift = beta - mean_c * scale
    return scale, shift


def _gn_lanes(srow, sqrow, gamma, beta):
    Cl = srow.shape[1]
    ng = Cl // 8
    gi = lax.broadcasted_iota(jnp.int32, (Cl, ng), 0) // 8
    gj = lax.broadcasted_iota(jnp.int32, (Cl, ng), 1)
    Gm = (gi == gj).astype(jnp.float32)
    denom = 8.0 * N_REAL
    gmean = _mm(srow, Gm, (((1,), (0,)))) / denom
    gmsq = _mm(sqrow, Gm, (((1,), (0,)))) / denom
    var = gmsq - gmean * gmean
    rstd = lax.rsqrt(var + EPS)
    mean_c = _mm(gmean, Gm, (((1,), (1,))))
    rstd_c = _mm(rstd, Gm, (((1,), (1,))))
    scale = rstd_c * gamma
    shift = beta - mean_c * scale
    return scale, shift


def _mask_lanes(n):
    col = n * TN + lax.broadcasted_iota(jnp.int32, (1, TN), 1)
    return (col < N_REAL).astype(jnp.float32)


def _acc_stats(ref, n, y, mask):
    s = jnp.sum(y * mask, axis=1, keepdims=True)
    q = jnp.sum(y * y * mask, axis=1, keepdims=True)
    contrib = jnp.concatenate([s, q], axis=1)

    @pl.when(n == 0)
    def _():
        ref[0] = jnp.zeros_like(ref[0])

    ref[0] += contrib


_SEQ = pltpu.CompilerParams(dimension_semantics=("arbitrary", "arbitrary"))
_SEQ1 = pltpu.CompilerParams(dimension_semantics=("arbitrary",))


def _k_enc(Wenc, img, b0):
    def body(w_ref, i_ref, b_ref, o_ref):
        o_ref[...] = _mm(w_ref[...], i_ref[...], (((1,), (1,)))) + b_ref[...]
    return pl.pallas_call(
        body, grid=(1,),
        in_specs=[pl.BlockSpec(Wenc.shape, lambda i: (0, 0)),
                  pl.BlockSpec(img.shape, lambda i: (0, 0)),
                  pl.BlockSpec((Wenc.shape[0], 1), lambda i: (0, 0))],
        out_specs=pl.BlockSpec((Wenc.shape[0], B), lambda i: (0, 0)),
        out_shape=jax.ShapeDtypeStruct((Wenc.shape[0], B), jnp.float32),
    )(Wenc, img, b0)


def _k_lin0(Wrv8, rv8, yenc):
    C = Wrv8.shape[0]

    def body(w_ref, r_ref, e_ref, o_ref, st_ref):
        n = pl.program_id(1)
        y = _mm(w_ref[...], r_ref[...], (((1,), (0,)))) + e_ref[0]
        o_ref[0] = y
        _acc_stats(st_ref, n, y, _mask_lanes(n))

    return pl.pallas_call(
        body, grid=(B, NT),
        in_specs=[pl.BlockSpec((C, 8), lambda b, n: (0, 0)),
                  pl.BlockSpec((8, TN), lambda b, n: (0, n)),
                  pl.BlockSpec((1, C, 1), lambda b, n: (b, 0, 0))],
        out_specs=[pl.BlockSpec((1, C, TN), lambda b, n: (b, 0, n)),
                   pl.BlockSpec((1, C, 2), lambda b, n: (b, 0, 0))],
        out_shape=[jax.ShapeDtypeStruct((B, C, NPAD), jnp.float32),
                   jax.ShapeDtypeStruct((B, C, 2), jnp.float32)],
        compiler_params=_SEQ,
    )(Wrv8, rv8, yenc)


def _k_a(x, xstats, gamma, beta, W, bias):
    Cin = W.shape[1]
    Cout = W.shape[0]

    def body(x_ref, st_ref, g_ref, be_ref, w_ref, b_ref, o_ref, ost_ref):
        n = pl.program_id(1)
        scale, shift = _gn_rows(st_ref[0], g_ref[...], be_ref[...])
        xn = jax.nn.relu(x_ref[0] * scale + shift)
        y = _mm(w_ref[...], xn, (((1,), (0,)))) + b_ref[...]
        o_ref[0] = y
        _acc_stats(ost_ref, n, y, _mask_lanes(n))

    return pl.pallas_call(
        body, grid=(B, NT),
        in_specs=[pl.BlockSpec((1, Cin, TN), lambda b, n: (b, 0, n)),
                  pl.BlockSpec((1, Cin, 2), lambda b, n: (b, 0, 0)),
                  pl.BlockSpec((Cin, 1), lambda b, n: (0, 0)),
                  pl.BlockSpec((Cin, 1), lambda b, n: (0, 0)),
                  pl.BlockSpec((Cout, Cin), lambda b, n: (0, 0)),
                  pl.BlockSpec((Cout, 1), lambda b, n: (0, 0))],
        out_specs=[pl.BlockSpec((1, Cout, TN), lambda b, n: (b, 0, n)),
                   pl.BlockSpec((1, Cout, 2), lambda b, n: (b, 0, 0))],
        out_shape=[jax.ShapeDtypeStruct((B, Cout, NPAD), jnp.float32),
                   jax.ShapeDtypeStruct((B, Cout, 2), jnp.float32)],
        compiler_params=_SEQ,
    )(x, xstats, gamma, beta, W, bias)


def _k_b(y1, y1stats, gamma, beta, Wc):
    def body(x_ref, st_ref, g_ref, be_ref, w_ref, o_ref):
        scale, shift = _gn_rows(st_ref[0], g_ref[...], be_ref[...])
        xn = jax.nn.relu(x_ref[0] * scale + shift)
        h = _mm(xn, w_ref[...], (((0,), (0,))))
        o_ref[0] = h[:, :CW]
        o_ref[1] = h[:, CW:]

    return pl.pallas_call(
        body, grid=(B, NT),
        in_specs=[pl.BlockSpec((1, CHALF, TN), lambda b, n: (b, 0, n)),
                  pl.BlockSpec((1, CHALF, 2), lambda b, n: (b, 0, 0)),
                  pl.BlockSpec((CHALF, 1), lambda b, n: (0, 0)),
                  pl.BlockSpec((CHALF, 1), lambda b, n: (0, 0)),
                  pl.BlockSpec((CHALF, CHALF), lambda b, n: (0, 0))],
        out_specs=pl.BlockSpec((2, TN, CW), lambda b, n: (b, n, 0)),
        out_shape=jax.ShapeDtypeStruct((NCHUNK, NPAD, CW), jnp.float32),
        compiler_params=_SEQ,
    )(y1, y1stats, gamma, beta, Wc)


def _k_stats(s):
    def body(s_ref, o_ref):
        n = pl.program_id(0)

        @pl.when(n == 0)
        def _():
            o_ref[...] = jnp.zeros_like(o_ref)

        for c in range(NCHUNK):
            sc = s_ref[c]
            o_ref[c, 0:1, :] += jnp.sum(sc, axis=0, keepdims=True)
            o_ref[c, 1:2, :] += jnp.sum(sc * sc, axis=0, keepdims=True)

    return pl.pallas_call(
        body, grid=(NT,),
        in_specs=[pl.BlockSpec((NCHUNK, TN, CW), lambda n: (0, n, 0))],
        out_specs=pl.BlockSpec((NCHUNK, 2, CW), lambda n: (0, 0, 0)),
        out_shape=jax.ShapeDtypeStruct((NCHUNK, 2, CW), jnp.float32),
        compiler_params=_SEQ1,
    )(s)


def _k_c(s, sstats, conv_b, gamma, beta, W2, b2, x, skip_W, skip_b):
    Cin = x.shape[1]
    Cout = W2.shape[0]
    has_skip = skip_W is not None

    def body(*refs):
        if has_skip:
            (s_ref, st_ref, cb_ref, g_ref, be_ref, w2_ref, b2_ref,
             x_ref, sw_ref, sb_ref, o_ref, ost_ref) = refs
        else:
            (s_ref, st_ref, cb_ref, g_ref, be_ref, w2_ref, b2_ref,
             x_ref, o_ref, ost_ref) = refs
        n = pl.program_id(1)
        z = b2_ref[...] * jnp.ones((1, TN), jnp.float32)
        for q in range(2):
            raw_s = st_ref[q, 0:1, :]
            raw_q = st_ref[q, 1:2, :]
            cb = cb_ref[q:q + 1, :]
            s_adj = raw_s + N_REAL * cb
            q_adj = raw_q + 2.0 * cb * raw_s + N_REAL * cb * cb
            scale, shift = _gn_lanes(s_adj, q_adj,
                                     g_ref[q:q + 1, :], be_ref[q:q + 1, :])
            sq = s_ref[q] + cb
            xq = jax.nn.relu(sq * scale + shift)
            z = z + _mm(w2_ref[:, q * CW:(q + 1) * CW], xq, (((1,), (1,))))
        if has_skip:
            z = z + _mm(sw_ref[...], x_ref[0], (((1,), (0,)))) + sb_ref[...]
        else:
            z = z + x_ref[0]
        o_ref[0] = z
        _acc_stats(ost_ref, n, z, _mask_lanes(n))

    in_arrays = [s, sstats, conv_b, gamma, beta, W2, b2, x]
    in_specs = [pl.BlockSpec((2, TN, CW), lambda b, n: (b, n, 0)),
                pl.BlockSpec((2, 2, CW), lambda b, n: (b, 0, 0)),
                pl.BlockSpec((2, CW), lambda b, n: (0, 0)),
                pl.BlockSpec((2, CW), lambda b, n: (0, 0)),
                pl.BlockSpec((2, CW), lambda b, n: (0, 0)),
                pl.BlockSpec((Cout, CHALF), lambda b, n: (0, 0)),
                pl.BlockSpec((Cout, 1), lambda b, n: (0, 0)),
                pl.BlockSpec((1, Cin, TN), lambda b, n: (b, 0, n))]
    if has_skip:
        in_arrays += [skip_W, skip_b]
        in_specs += [pl.BlockSpec((Cout, Cin), lambda b, n: (0, 0)),
                     pl.BlockSpec((Cout, 1), lambda b, n: (0, 0))]

    return pl.pallas_call(
        body, grid=(B, NT),
        in_specs=in_specs,
        out_specs=[pl.BlockSpec((1, Cout, TN), lambda b, n: (b, 0, n)),
                   pl.BlockSpec((1, Cout, 2), lambda b, n: (b, 0, 0))],
        out_shape=[jax.ShapeDtypeStruct((B, Cout, NPAD), jnp.float32),
                   jax.ShapeDtypeStruct((B, Cout, 2), jnp.float32)],
        compiler_params=_SEQ,
    )(*in_arrays)


def _k_d(y, W1, b1, W2, b2):
    C1 = W1.shape[0]
    C2 = W2.shape[0]
    Cin = W1.shape[1]

    def body(y_ref, w1_ref, b1_ref, w2_ref, b2_ref, o_ref, ost_ref):
        n = pl.program_id(1)
        t = jax.nn.relu(_mm(w1_ref[...], y_ref[0], (((1,), (0,)))) + b1_ref[...])
        u = _mm(w2_ref[...], t, (((1,), (0,)))) + b2_ref[...]
        o_ref[0] = u
        _acc_stats(ost_ref, n, u, _mask_lanes(n))

    return pl.pallas_call(
        body, grid=(B, NT),
        in_specs=[pl.BlockSpec((1, Cin, TN), lambda b, n: (b, 0, n)),
                  pl.BlockSpec((C1, Cin), lambda b, n: (0, 0)),
                  pl.BlockSpec((C1, 1), lambda b, n: (0, 0)),
                  pl.BlockSpec((C2, C1), lambda b, n: (0, 0)),
                  pl.BlockSpec((C2, 1), lambda b, n: (0, 0))],
        out_specs=[pl.BlockSpec((1, C2, TN), lambda b, n: (b, 0, n)),
                   pl.BlockSpec((1, C2, 2), lambda b, n: (b, 0, 0))],
        out_shape=[jax.ShapeDtypeStruct((B, C2, NPAD), jnp.float32),
                   jax.ShapeDtypeStruct((B, C2, 2), jnp.float32)],
        compiler_params=_SEQ,
    )(y, W1, b1, W2, b2)


def _k_e(u, ustats, gamma, beta, W3, b3):
    Cin = W3.shape[1]
    Cout = W3.shape[0]

    def body(u_ref, st_ref, g_ref, be_ref, w_ref, b_ref, o_ref):
        scale, shift = _gn_rows(st_ref[0], g_ref[...], be_ref[...])
        xn = jax.nn.relu(u_ref[0] * scale + shift)
        o_ref[0] = _mm(w_ref[...], xn, (((1,), (0,)))) + b_ref[...]

    return pl.pallas_call(
        body, grid=(B, NT),
        in_specs=[pl.BlockSpec((1, Cin, TN), lambda b, n: (b, 0, n)),
                  pl.BlockSpec((1, Cin, 2), lambda b, n: (b, 0, 0)),
                  pl.BlockSpec((Cin, 1), lambda b, n: (0, 0)),
                  pl.BlockSpec((Cin, 1), lambda b, n: (0, 0)),
                  pl.BlockSpec((Cout, Cin), lambda b, n: (0, 0)),
                  pl.BlockSpec((Cout, 1), lambda b, n: (0, 0))],
        out_specs=pl.BlockSpec((1, Cout, TN), lambda b, n: (b, 0, n)),
        out_shape=jax.ShapeDtypeStruct((B, Cout, NPAD), jnp.float32),
        compiler_params=_SEQ,
    )(u, ustats, gamma, beta, W3, b3)


RING = 2


def _spmm_sc(h4, src4, dstv, w16, n_batches):
    NB = n_batches
    epad = dstv.shape[0]

    mesh = plsc.VectorSubcoreMesh(core_axis_name="c", subcore_axis_name="s",
                                  num_cores=NSC, num_subcores=NSUB)

    @functools.partial(
        pl.kernel, mesh=mesh,
        out_type=jax.ShapeDtypeStruct((NCHUNK * NPAD, CW), jnp.float32),
        scratch_types=(
            [pltpu.VMEM((G,), jnp.int32) for _ in range(RING)]
            + [pltpu.VMEM((G,), jnp.int32) for _ in range(RING)]
            + [pltpu.VMEM((G, 16), jnp.float32) for _ in range(RING)]
            + [pltpu.VMEM((G, CW), jnp.float32) for _ in range(RING)]
            + [pltpu.VMEM((8, CW), jnp.float32),
               pltpu.VMEM_SHARED((NPAD, CW), jnp.float32)]
            + [pltpu.SemaphoreType.DMA] * (2 * RING)
        ),
    )
    def k(h4_hbm, src4_hbm, dstv_hbm, w16_hbm, out_hbm, *scr):
        idx_v = scr[:RING]
        dst_v = scr[RING:2 * RING]
        w_v = scr[2 * RING:3 * RING]
        rows_v = scr[3 * RING:4 * RING]
        zero_v = scr[4 * RING]
        acc = scr[4 * RING + 1]
        sem_g = scr[4 * RING + 2:4 * RING + 2 + RING]
        sem_s = scr[4 * RING + 2 + RING:]
        core = lax.axis_index("c")
        sub = lax.axis_index("s")
        zvec = jnp.zeros((16,), jnp.float32)
        for r in range(8):
            for kk in range(CW // 16):
                zero_v[r, pl.ds(kk * 16, 16)] = zvec

        def scale(r):
            def g_body(g, _):
                wv = w_v[r][g, :]
                for kk in range(CW // 16):
                    sl = pl.ds(kk * 16, 16)
                    rows_v[r][g, sl] = rows_v[r][g, sl] * wv
                return 0
            lax.fori_loop(0, G, g_body, 0)

        def load_issue(chunk, b):
            r = b % RING
            off = sub * NB * G + b * G
            pltpu.sync_copy(src4_hbm.at[pl.ds(chunk * epad + off, G)], idx_v[r])
            pltpu.sync_copy(dstv_hbm.at[pl.ds(off, G)], dst_v[r])
            pltpu.sync_copy(w16_hbm.at[pl.ds(off, G)], w_v[r])
            return pltpu.async_copy(h4_hbm.at[idx_v[r]], rows_v[r], sem_g[r])

        def chunk_body(chunk, _):
            for z in range(RPT // 8):
                pltpu.sync_copy(zero_v, acc.at[pl.ds(sub * RPT + z * 8, 8)])
            plsc.subcore_barrier()

            gd = [None] * RING
            sd = [None] * RING
            gd[0] = load_issue(chunk, 0)
            for b in range(NB):
                r = b % RING
                nxt = (b + 1) % RING
                if b + 1 < NB:
                    if sd[nxt] is not None:
                        sd[nxt].wait()
                    gd[nxt] = load_issue(chunk, b + 1)
                gd[r].wait()
                scale(r)
                sd[r] = pltpu.async_copy(rows_v[r], acc.at[dst_v[r]],
                                         sem_s[r], add=True)
            for r in range(RING):
                if sd[r] is not None:
                    sd[r].wait()
            plsc.subcore_barrier()
            pltpu.sync_copy(
                acc.at[pl.ds(sub * RPT, RPT)],
                out_hbm.at[pl.ds(chunk * NPAD + sub * RPT, RPT)])
            plsc.subcore_barrier()
            return 0

        lo = core * (NCHUNK // NSC)
        lax.fori_loop(lo, lo + NCHUNK // NSC, chunk_body, 0)

    return k(h4, src4, dstv, w16)


def kernel(image_resnet, params, ref_vertices, edge_index, edge_weight):
    N = ref_vertices.shape[1]
    f32 = jnp.float32

    src = edge_index[0]
    dst = edge_index[1]
    E = src.shape[0]
    epad = ((E + NSUB * G - 1) // (NSUB * G)) * (NSUB * G)
    n_batches = epad // (NSUB * G)
    pad = epad - E
    src_p = jnp.pad(src, (0, pad))
    dst_p = jnp.pad(dst, (0, pad))
    w_p = jnp.pad(edge_weight, (0, pad))
    src4 = (src_p[None, :]
            + NPAD * jnp.arange(NCHUNK, dtype=jnp.int32)[:, None]).reshape(-1)
    dst2 = dst_p
    w16 = jnp.broadcast_to(w_p[:, None], (epad, 16))

    W0 = params['lin0_W']
    rv8 = jnp.pad(ref_vertices, ((0, 5), (0, NPAD - N)))
    Wrv8 = jnp.pad(W0[:, :3], ((0, 0), (0, 5)))
    yenc = _k_enc(W0[:, 3:], image_resnet, params['lin0_b'][:, None])
    yenc = jnp.transpose(yenc)[:, :, None]
    x, xstats = _k_lin0(Wrv8, rv8, yenc)

    for p in params['blocks']:
        cv = lambda a: a[:, None].astype(f32)
        y1, y1stats = _k_a(x, xstats, cv(p['pre_g']), cv(p['pre_b']),
                           p['lin1_W'], cv(p['lin1_b']))
        h = _k_b(y1, y1stats, cv(p['n1_g']), cv(p['n1_b']), p['conv_W'])
        s4 = _spmm_sc(h.reshape(NCHUNK * NPAD, CW), src4, dst2, w16,
                      n_batches)
        s = s4.reshape(NCHUNK, NPAD, CW)
        sstats = _k_stats(s)
        x, xstats = _k_c(s, sstats, p['conv_b'].reshape(2, CW),
                         p['n2_g'].reshape(2, CW), p['n2_b'].reshape(2, CW),
                         p['lin2_W'], cv(p['lin2_b']), x,
                         p.get('skip_W'), cv(p['skip_b']) if 'skip_W' in p else None)

    u, ustats = _k_d(x, params['shape_W1'], cv(params['shape_b1']),
                     params['shape_W2'], cv(params['shape_b2']))
    out = _k_e(u, ustats, cv(params['shape_ng']), cv(params['shape_nb']),
               params['shape_W3'], cv(params['shape_b3']))
    return out[:, :, :N]


# speedup vs baseline: 2.0637x; 1.8090x over previous
import functools
import jax
import jax.numpy as jnp
from jax import lax
from jax.experimental import pallas as pl
from jax.experimental.pallas import tpu as pltpu
from jax.experimental.pallas import tpu_sc as plsc

N_REAL = 6890
NPAD = 6912
B = 4
CHALF = 256
ROW = B * CHALF
NCHUNK = 8
CW = ROW // NCHUNK
NSC = 2
NSUB = 16
G = 128
RPT = NPAD // NSUB
TN = 768
NT = NPAD // TN
EPS = 1e-5

HI = lax.Precision.DEFAULT


def _mm(a, b, dims):
    return lax.dot_general(a, b, (dims, ((), ())), precision=HI,
                           preferred_element_type=jnp.float32)


def _gn_rows(stats, gamma, beta):
    C = stats.shape[0]
    ng = C // 8
    gi = lax.broadcasted_iota(jnp.int32, (ng, C), 1) // 8
    gj = lax.broadcasted_iota(jnp.int32, (ng, C), 0)
    Gm = (gi == gj).astype(jnp.float32)
    gs = _mm(Gm, stats, (((1,), (0,))))
    denom = 8.0 * N_REAL
    mean = gs[:, 0:1] / denom
    var = gs[:, 1:2] / denom - mean * mean
    rstd = lax.rsqrt(var + EPS)
    mean_c = _mm(Gm, mean, (((0,), (0,))))
    rstd_c = _mm(Gm, rstd, (((0,), (0,))))
    scale = rstd_c * gamma
    shift = beta - mean_c * scale
    return scale, shift


def _gn_lanes(srow, sqrow, gamma, beta):
    Cl = srow.shape[1]
    ng = Cl // 8
    gi = lax.broadcasted_iota(jnp.int32, (Cl, ng), 0) // 8
    gj = lax.broadcasted_iota(jnp.int32, (Cl, ng), 1)
    Gm = (gi == gj).astype(jnp.float32)
    denom = 8.0 * N_REAL
    gmean = _mm(srow, Gm, (((1,), (0,)))) / denom
    gmsq = _mm(sqrow, Gm, (((1,), (0,)))) / denom
    var = gmsq - gmean * gmean
    rstd = lax.rsqrt(var + EPS)
    mean_c = _mm(gmean, Gm, (((1,), (1,))))
    rstd_c = _mm(rstd, Gm, (((1,), (1,))))
    scale = rstd_c * gamma
    shift = beta - mean_c * scale
    return scale, shift


def _mask_lanes(n):
    col = n * TN + lax.broadcasted_iota(jnp.int32, (1, TN), 1)
    return (col < N_REAL).astype(jnp.float32)


def _acc_stats(ref, n, y, mask):
    s = jnp.sum(y * mask, axis=1, keepdims=True)
    q = jnp.sum(y * y * mask, axis=1, keepdims=True)
    contrib = jnp.concatenate([s, q], axis=1)

    @pl.when(n == 0)
    def _():
        ref[0] = jnp.zeros_like(ref[0])

    ref[0] += contrib


_SEQ = pltpu.CompilerParams(dimension_semantics=("arbitrary", "arbitrary"))
_SEQ1 = pltpu.CompilerParams(dimension_semantics=("arbitrary",))


def _k_enc(Wenc, img, b0):
    def body(w_ref, i_ref, b_ref, o_ref):
        o_ref[...] = _mm(w_ref[...], i_ref[...], (((1,), (1,)))) + b_ref[...]
    return pl.pallas_call(
        body, grid=(1,),
        in_specs=[pl.BlockSpec(Wenc.shape, lambda i: (0, 0)),
                  pl.BlockSpec(img.shape, lambda i: (0, 0)),
                  pl.BlockSpec((Wenc.shape[0], 1), lambda i: (0, 0))],
        out_specs=pl.BlockSpec((Wenc.shape[0], B), lambda i: (0, 0)),
        out_shape=jax.ShapeDtypeStruct((Wenc.shape[0], B), jnp.float32),
    )(Wenc, img, b0)


def _k_lin0(Wrv8, rv8, yenc):
    C = Wrv8.shape[0]

    def body(w_ref, r_ref, e_ref, o_ref, st_ref):
        n = pl.program_id(1)
        y = _mm(w_ref[...], r_ref[...], (((1,), (0,)))) + e_ref[0]
        o_ref[0] = y
        _acc_stats(st_ref, n, y, _mask_lanes(n))

    return pl.pallas_call(
        body, grid=(B, NT),
        in_specs=[pl.BlockSpec((C, 8), lambda b, n: (0, 0)),
                  pl.BlockSpec((8, TN), lambda b, n: (0, n)),
                  pl.BlockSpec((1, C, 1), lambda b, n: (b, 0, 0))],
        out_specs=[pl.BlockSpec((1, C, TN), lambda b, n: (b, 0, n)),
                   pl.BlockSpec((1, C, 2), lambda b, n: (b, 0, 0))],
        out_shape=[jax.ShapeDtypeStruct((B, C, NPAD), jnp.float32),
                   jax.ShapeDtypeStruct((B, C, 2), jnp.float32)],
        compiler_params=_SEQ,
    )(Wrv8, rv8, yenc)


def _k_a(x, xstats, gamma, beta, W, bias):
    Cin = W.shape[1]
    Cout = W.shape[0]

    def body(x_ref, st_ref, g_ref, be_ref, w_ref, b_ref, o_ref, ost_ref):
        n = pl.program_id(1)
        scale, shift = _gn_rows(st_ref[0], g_ref[...], be_ref[...])
        xn = jax.nn.relu(x_ref[0] * scale + shift)
        y = _mm(w_ref[...], xn, (((1,), (0,)))) + b_ref[...]
        o_ref[0] = y
        _acc_stats(ost_ref, n, y, _mask_lanes(n))

    return pl.pallas_call(
        body, grid=(B, NT),
        in_specs=[pl.BlockSpec((1, Cin, TN), lambda b, n: (b, 0, n)),
                  pl.BlockSpec((1, Cin, 2), lambda b, n: (b, 0, 0)),
                  pl.BlockSpec((Cin, 1), lambda b, n: (0, 0)),
                  pl.BlockSpec((Cin, 1), lambda b, n: (0, 0)),
                  pl.BlockSpec((Cout, Cin), lambda b, n: (0, 0)),
                  pl.BlockSpec((Cout, 1), lambda b, n: (0, 0))],
        out_specs=[pl.BlockSpec((1, Cout, TN), lambda b, n: (b, 0, n)),
                   pl.BlockSpec((1, Cout, 2), lambda b, n: (b, 0, 0))],
        out_shape=[jax.ShapeDtypeStruct((B, Cout, NPAD), jnp.float32),
                   jax.ShapeDtypeStruct((B, Cout, 2), jnp.float32)],
        compiler_params=_SEQ,
    )(x, xstats, gamma, beta, W, bias)


def _k_b(y1, y1stats, gamma, beta, Wc):
    def body(x_ref, st_ref, g_ref, be_ref, w_ref, o_ref):
        scale, shift = _gn_rows(st_ref[0], g_ref[...], be_ref[...])
        xn = jax.nn.relu(x_ref[0] * scale + shift)
        h = _mm(xn, w_ref[...], (((0,), (0,))))
        o_ref[0] = h[:, :CW]
        o_ref[1] = h[:, CW:]

    return pl.pallas_call(
        body, grid=(B, NT),
        in_specs=[pl.BlockSpec((1, CHALF, TN), lambda b, n: (b, 0, n)),
                  pl.BlockSpec((1, CHALF, 2), lambda b, n: (b, 0, 0)),
                  pl.BlockSpec((CHALF, 1), lambda b, n: (0, 0)),
                  pl.BlockSpec((CHALF, 1), lambda b, n: (0, 0)),
                  pl.BlockSpec((CHALF, CHALF), lambda b, n: (0, 0))],
        out_specs=pl.BlockSpec((2, TN, CW), lambda b, n: (b, n, 0)),
        out_shape=jax.ShapeDtypeStruct((NCHUNK, NPAD, CW), jnp.float32),
        compiler_params=_SEQ,
    )(y1, y1stats, gamma, beta, Wc)


def _k_stats(s):
    def body(s_ref, o_ref):
        n = pl.program_id(0)

        @pl.when(n == 0)
        def _():
            o_ref[...] = jnp.zeros_like(o_ref)

        for c in range(NCHUNK):
            sc = s_ref[c]
            o_ref[c, 0:1, :] += jnp.sum(sc, axis=0, keepdims=True)
            o_ref[c, 1:2, :] += jnp.sum(sc * sc, axis=0, keepdims=True)

    return pl.pallas_call(
        body, grid=(NT,),
        in_specs=[pl.BlockSpec((NCHUNK, TN, CW), lambda n: (0, n, 0))],
        out_specs=pl.BlockSpec((NCHUNK, 2, CW), lambda n: (0, 0, 0)),
        out_shape=jax.ShapeDtypeStruct((NCHUNK, 2, CW), jnp.float32),
        compiler_params=_SEQ1,
    )(s)


def _k_c(s, sstats, conv_b, gamma, beta, W2, b2, x, skip_W, skip_b):
    Cin = x.shape[1]
    Cout = W2.shape[0]
    has_skip = skip_W is not None

    def body(*refs):
        if has_skip:
            (s_ref, st_ref, cb_ref, g_ref, be_ref, w2_ref, b2_ref,
             x_ref, sw_ref, sb_ref, o_ref, ost_ref) = refs
        else:
            (s_ref, st_ref, cb_ref, g_ref, be_ref, w2_ref, b2_ref,
             x_ref, o_ref, ost_ref) = refs
        n = pl.program_id(1)
        z = b2_ref[...] * jnp.ones((1, TN), jnp.float32)
        for q in range(2):
            raw_s = st_ref[q, 0:1, :]
            raw_q = st_ref[q, 1:2, :]
            cb = cb_ref[q:q + 1, :]
            s_adj = raw_s + N_REAL * cb
            q_adj = raw_q + 2.0 * cb * raw_s + N_REAL * cb * cb
            scale, shift = _gn_lanes(s_adj, q_adj,
                                     g_ref[q:q + 1, :], be_ref[q:q + 1, :])
            sq = s_ref[q] + cb
            xq = jax.nn.relu(sq * scale + shift)
            z = z + _mm(w2_ref[:, q * CW:(q + 1) * CW], xq, (((1,), (1,))))
        if has_skip:
            z = z + _mm(sw_ref[...], x_ref[0], (((1,), (0,)))) + sb_ref[...]
        else:
            z = z + x_ref[0]
        o_ref[0] = z
        _acc_stats(ost_ref, n, z, _mask_lanes(n))

    in_arrays = [s, sstats, conv_b, gamma, beta, W2, b2, x]
    in_specs = [pl.BlockSpec((2, TN, CW), lambda b, n: (b, n, 0)),
                pl.BlockSpec((2, 2, CW), lambda b, n: (b, 0, 0)),
                pl.BlockSpec((2, CW), lambda b, n: (0, 0)),
                pl.BlockSpec((2, CW), lambda b, n: (0, 0)),
                pl.BlockSpec((2, CW), lambda b, n: (0, 0)),
                pl.BlockSpec((Cout, CHALF), lambda b, n: (0, 0)),
                pl.BlockSpec((Cout, 1), lambda b, n: (0, 0)),
                pl.BlockSpec((1, Cin, TN), lambda b, n: (b, 0, n))]
    if has_skip:
        in_arrays += [skip_W, skip_b]
        in_specs += [pl.BlockSpec((Cout, Cin), lambda b, n: (0, 0)),
                     pl.BlockSpec((Cout, 1), lambda b, n: (0, 0))]

    return pl.pallas_call(
        body, grid=(B, NT),
        in_specs=in_specs,
        out_specs=[pl.BlockSpec((1, Cout, TN), lambda b, n: (b, 0, n)),
                   pl.BlockSpec((1, Cout, 2), lambda b, n: (b, 0, 0))],
        out_shape=[jax.ShapeDtypeStruct((B, Cout, NPAD), jnp.float32),
                   jax.ShapeDtypeStruct((B, Cout, 2), jnp.float32)],
        compiler_params=_SEQ,
    )(*in_arrays)


def _k_d(y, W1, b1, W2, b2):
    C1 = W1.shape[0]
    C2 = W2.shape[0]
    Cin = W1.shape[1]

    def body(y_ref, w1_ref, b1_ref, w2_ref, b2_ref, o_ref, ost_ref):
        n = pl.program_id(1)
        t = jax.nn.relu(_mm(w1_ref[...], y_ref[0], (((1,), (0,)))) + b1_ref[...])
        u = _mm(w2_ref[...], t, (((1,), (0,)))) + b2_ref[...]
        o_ref[0] = u
        _acc_stats(ost_ref, n, u, _mask_lanes(n))

    return pl.pallas_call(
        body, grid=(B, NT),
        in_specs=[pl.BlockSpec((1, Cin, TN), lambda b, n: (b, 0, n)),
                  pl.BlockSpec((C1, Cin), lambda b, n: (0, 0)),
                  pl.BlockSpec((C1, 1), lambda b, n: (0, 0)),
                  pl.BlockSpec((C2, C1), lambda b, n: (0, 0)),
                  pl.BlockSpec((C2, 1), lambda b, n: (0, 0))],
        out_specs=[pl.BlockSpec((1, C2, TN), lambda b, n: (b, 0, n)),
                   pl.BlockSpec((1, C2, 2), lambda b, n: (b, 0, 0))],
        out_shape=[jax.ShapeDtypeStruct((B, C2, NPAD), jnp.float32),
                   jax.ShapeDtypeStruct((B, C2, 2), jnp.float32)],
        compiler_params=_SEQ,
    )(y, W1, b1, W2, b2)


def _k_e(u, ustats, gamma, beta, W3, b3):
    Cin = W3.shape[1]
    Cout = W3.shape[0]

    def body(u_ref, st_ref, g_ref, be_ref, w_ref, b_ref, o_ref):
        scale, shift = _gn_rows(st_ref[0], g_ref[...], be_ref[...])
        xn = jax.nn.relu(u_ref[0] * scale + shift)
        o_ref[0] = _mm(w_ref[...], xn, (((1,), (0,)))) + b_ref[...]

    return pl.pallas_call(
        body, grid=(B, NT),
        in_specs=[pl.BlockSpec((1, Cin, TN), lambda b, n: (b, 0, n)),
                  pl.BlockSpec((1, Cin, 2), lambda b, n: (b, 0, 0)),
                  pl.BlockSpec((Cin, 1), lambda b, n: (0, 0)),
                  pl.BlockSpec((Cin, 1), lambda b, n: (0, 0)),
                  pl.BlockSpec((Cout, Cin), lambda b, n: (0, 0)),
                  pl.BlockSpec((Cout, 1), lambda b, n: (0, 0))],
        out_specs=pl.BlockSpec((1, Cout, TN), lambda b, n: (b, 0, n)),
        out_shape=jax.ShapeDtypeStruct((B, Cout, NPAD), jnp.float32),
        compiler_params=_SEQ,
    )(u, ustats, gamma, beta, W3, b3)


RING = 2


def _spmm_sc(h4, src4, dstv, w16, n_batches):
    NB = n_batches
    epad = dstv.shape[0]

    mesh = plsc.VectorSubcoreMesh(core_axis_name="c", subcore_axis_name="s",
                                  num_cores=NSC, num_subcores=NSUB)

    @functools.partial(
        pl.kernel, mesh=mesh,
        out_type=jax.ShapeDtypeStruct((NCHUNK * NPAD, CW), jnp.float32),
        scratch_types=(
            [pltpu.VMEM((G,), jnp.int32) for _ in range(3)]
            + [pltpu.VMEM((G,), jnp.int32) for _ in range(3)]
            + [pltpu.VMEM((G // 8, CW), jnp.float32) for _ in range(3)]
            + [pltpu.VMEM((G, CW), jnp.float32) for _ in range(RING)]
            + [pltpu.VMEM((8, CW), jnp.float32),
               pltpu.VMEM_SHARED((NPAD, CW), jnp.float32)]
            + [pltpu.SemaphoreType.DMA] * (3 + 2 * RING)
        ),
    )
    def k(h4_hbm, src4_hbm, dstv_hbm, w16_hbm, out_hbm, *scr):
        idx_v = scr[:3]
        dst_v = scr[3:6]
        w_v = scr[6:9]
        rows_v = scr[9:9 + RING]
        zero_v = scr[9 + RING]
        acc = scr[10 + RING]
        sem_i = scr[11 + RING:14 + RING]
        sem_g = scr[14 + RING:14 + 2 * RING]
        sem_s = scr[14 + 2 * RING:]
        core = lax.axis_index("c")
        sub = lax.axis_index("s")
        zvec = jnp.zeros((16,), jnp.float32)
        for r in range(8):
            for kk in range(CW // 16):
                zero_v[r, pl.ds(kk * 16, 16)] = zvec

        def scale(r2, r3):
            def g_body(g, _):
                wv = w_v[r3][g // 8, pl.ds((g % 8) * 16, 16)]
                for kk in range(CW // 16):
                    sl = pl.ds(kk * 16, 16)
                    rows_v[r2][g, sl] = rows_v[r2][g, sl] * wv
                return 0
            lax.fori_loop(0, G, g_body, 0)

        def load(chunk, b):
            r = b % 3
            off = sub * NB * G + b * G
            d1 = pltpu.async_copy(src4_hbm.at[pl.ds(chunk * epad + off, G)],
                                  idx_v[r], sem_i[r])
            d2 = pltpu.async_copy(dstv_hbm.at[pl.ds(off, G)], dst_v[r],
                                  sem_i[r])
            d3 = pltpu.async_copy(
                w16_hbm.at[pl.ds((sub * NB + b) * (G // 8), G // 8)],
                w_v[r], sem_i[r])
            return (d1, d2, d3)

        def chunk_body(chunk, _):
            for z in range(RPT // 8):
                pltpu.sync_copy(zero_v, acc.at[pl.ds(sub * RPT + z * 8, 8)])
            plsc.subcore_barrier()

            ld = [None] * 3
            gd = [None] * RING
            sd = [None] * RING
            ld[0] = load(chunk, 0)
            if NB > 1:
                ld[1] = load(chunk, 1)
            for d in ld[0]:
                d.wait()
            ld[0] = None
            gd[0] = pltpu.async_copy(h4_hbm.at[idx_v[0]], rows_v[0], sem_g[0])
            for b in range(NB):
                r2 = b % RING
                r3 = b % 3
                if sd[(b + 1) % RING] is not None:
                    sd[(b + 1) % RING].wait()
                    sd[(b + 1) % RING] = None
                if b + 2 < NB:
                    ld[(b + 2) % 3] = load(chunk, b + 2)
                if b + 1 < NB:
                    for d in ld[(b + 1) % 3]:
                        d.wait()
                    ld[(b + 1) % 3] = None
                    gd[(b + 1) % RING] = pltpu.async_copy(
                        h4_hbm.at[idx_v[(b + 1) % 3]],
                        rows_v[(b + 1) % RING], sem_g[(b + 1) % RING])
                gd[r2].wait()
                scale(r2, r3)
                sd[r2] = pltpu.async_copy(rows_v[r2], acc.at[dst_v[r3]],
                                          sem_s[r2], add=True)
            for r in range(RING):
                if sd[r] is not None:
                    sd[r].wait()
            plsc.subcore_barrier()
            pltpu.sync_copy(
                acc.at[pl.ds(sub * RPT, RPT)],
                out_hbm.at[pl.ds(chunk * NPAD + sub * RPT, RPT)])
            plsc.subcore_barrier()
            return 0

        lo = core * (NCHUNK // NSC)
        lax.fori_loop(lo, lo + NCHUNK // NSC, chunk_body, 0)

    return k(h4, src4, dstv, w16)


def kernel(image_resnet, params, ref_vertices, edge_index, edge_weight):
    N = ref_vertices.shape[1]
    f32 = jnp.float32

    src = edge_index[0]
    dst = edge_index[1]
    E = src.shape[0]
    epad = ((E + NSUB * G - 1) // (NSUB * G)) * (NSUB * G)
    n_batches = epad // (NSUB * G)
    pad = epad - E
    src_p = jnp.pad(src, (0, pad))
    dst_p = jnp.pad(dst, (0, pad))
    w_p = jnp.pad(edge_weight, (0, pad))
    src4 = (src_p[None, :]
            + NPAD * jnp.arange(NCHUNK, dtype=jnp.int32)[:, None]).reshape(-1)
    dst2 = dst_p
    w16 = jnp.broadcast_to(w_p[:, None], (epad, 16)).reshape(-1, CW)

    W0 = params['lin0_W']
    rv8 = jnp.pad(ref_vertices, ((0, 5), (0, NPAD - N)))
    Wrv8 = jnp.pad(W0[:, :3], ((0, 0), (0, 5)))
    yenc = _k_enc(W0[:, 3:], image_resnet, params['lin0_b'][:, None])
    yenc = jnp.transpose(yenc)[:, :, None]
    x, xstats = _k_lin0(Wrv8, rv8, yenc)

    for p in params['blocks']:
        cv = lambda a: a[:, None].astype(f32)
        y1, y1stats = _k_a(x, xstats, cv(p['pre_g']), cv(p['pre_b']),
                           p['lin1_W'], cv(p['lin1_b']))
        h = _k_b(y1, y1stats, cv(p['n1_g']), cv(p['n1_b']), p['conv_W'])
        s4 = _spmm_sc(h.reshape(NCHUNK * NPAD, CW), src4, dst2, w16,
                      n_batches)
        s = s4.reshape(NCHUNK, NPAD, CW)
        sstats = _k_stats(s)
        x, xstats = _k_c(s, sstats, p['conv_b'].reshape(2, CW),
                         p['n2_g'].reshape(2, CW), p['n2_b'].reshape(2, CW),
                         p['lin2_W'], cv(p['lin2_b']), x,
                         p.get('skip_W'), cv(p['skip_b']) if 'skip_W' in p else None)

    u, ustats = _k_d(x, params['shape_W1'], cv(params['shape_b1']),
                     params['shape_W2'], cv(params['shape_b2']))
    out = _k_e(u, ustats, cv(params['shape_ng']), cv(params['shape_nb']),
               params['shape_W3'], cv(params['shape_b3']))
    return out[:, :, :N]
